# baseline (speedup 1.0000x reference)
import functools

import numpy as np
import jax
import jax.numpy as jnp
from jax import lax
from jax.experimental import pallas as pl
from jax.experimental.pallas import tpu as pltpu
from jax.experimental.pallas import tpu_sc as plsc

NGH = 7
SUBQ = -8
SUBD = 1
POS_D = 3
NEG_D = 5
BORDER = 16
SUBD_NEG = -8
B, C, H, W = 2, 128, 384, 384
HW = H * W

def _make_offsets():
    pos, neg = [], []
    pos_d2, neg_d2, rad2 = POS_D ** 2, NEG_D ** 2, NGH ** 2
    rad = (NGH // SUBD) * NGH
    for j in range(-rad, rad + 1, SUBD):
        for i in range(-rad, rad + 1, SUBD):
            d2 = i * i + j * j
            if d2 <= pos_d2:
                pos.append((i, j))
            elif neg_d2 <= d2 <= rad2:
                neg.append((i, j))
    return (np.array(pos, dtype=np.int32).T, np.array(neg, dtype=np.int32).T)

_POS_OFF, _NEG_OFF = _make_offsets()
NPOS = _POS_OFF.shape[1]
NNEG = _NEG_OFF.shape[1]
NOFF = NPOS + NNEG
NOFF_PAD = 112
_ALL_OFF = np.concatenate(
    [_POS_OFF, _NEG_OFF, np.zeros((2, NOFF_PAD - NOFF), np.int32)], axis=1)

NQ = (H - 2 * BORDER) * (W - 2 * BORDER) // (SUBQ * SUBQ)
N = B * NQ
NPAD = 4096

_NC, _NS = 2, 16
_NW = _NC * _NS
_QPW = N // _NW
_GPW = NPAD // _NW

_sc_mesh = plsc.VectorSubcoreMesh(
    core_axis_name="c", subcore_axis_name="s", num_cores=_NC, num_subcores=_NS)


def _rand_grid(key):
    kx, ky = jax.random.split(key)
    x = jax.random.randint(kx, (NQ,), BORDER, W - BORDER)
    y = jax.random.randint(ky, (NQ,), BORDER, H - BORDER)
    x = jnp.broadcast_to(x[None, :], (B, NQ)).reshape(-1)
    y = jnp.broadcast_to(y[None, :], (B, NQ)).reshape(-1)
    b = jnp.broadcast_to(jnp.arange(B)[:, None], (B, NQ)).reshape(-1)
    return b, y, x


@functools.partial(
    pl.kernel,
    out_type=(
        jax.ShapeDtypeStruct((NPAD, C), jnp.float32),
        jax.ShapeDtypeStruct((NPAD, C), jnp.float32),
    ),
    mesh=_sc_mesh,
    compiler_params=pltpu.CompilerParams(needs_layout_passes=False),
    scratch_types=[
        pltpu.VMEM((_GPW,), jnp.int32),
        pltpu.VMEM((_GPW, C), jnp.float32),
        pltpu.SemaphoreType.DMA,
    ],
)
def _sc_gather_rows(f1t_hbm, f2t_hbm, qidx_hbm, didx_hbm,
                    f1g_hbm, dg_hbm, idx_v, rows_v, sem):
    wid = lax.axis_index("s") * _NC + lax.axis_index("c")
    base = wid * _GPW
    pltpu.sync_copy(qidx_hbm.at[pl.ds(base, _GPW)], idx_v)
    pltpu.async_copy(f1t_hbm.at[idx_v], rows_v, sem).wait()
    pltpu.sync_copy(rows_v, f1g_hbm.at[pl.ds(base, _GPW)])
    pltpu.sync_copy(didx_hbm.at[pl.ds(base, _GPW)], idx_v)
    pltpu.async_copy(f2t_hbm.at[idx_v], rows_v, sem).wait()
    pltpu.sync_copy(rows_v, dg_hbm.at[pl.ds(base, _GPW)])


@functools.partial(
    pl.kernel,
    out_type=jax.ShapeDtypeStruct((N * NOFF_PAD,), jnp.float32),
    mesh=_sc_mesh,
    compiler_params=pltpu.CompilerParams(needs_layout_passes=False),
    scratch_types=[
        pltpu.VMEM((_QPW * NOFF_PAD,), jnp.int32),
        pltpu.VMEM((_QPW * C,), jnp.float32),
        pltpu.VMEM((NOFF_PAD, C), jnp.float32),
        pltpu.VMEM((NOFF_PAD, C), jnp.float32),
        pltpu.VMEM((NOFF_PAD * 16,), jnp.float32),
        pltpu.VMEM((_QPW * NOFF_PAD,), jnp.float32),
        pltpu.SemaphoreType.DMA,
        pltpu.SemaphoreType.DMA,
    ],
)
def _sc_ngh_scores(f2t_hbm, f1g_hbm, nidx_hbm, ns_hbm,
                   nidx_v, f1_v, rows0_v, rows1_v, accs_v, out_v, sem0, sem1):
    wid = lax.axis_index("s") * _NC + lax.axis_index("c")

    pltpu.sync_copy(nidx_hbm.at[pl.ds(wid * (_QPW * NOFF_PAD), _QPW * NOFF_PAD)],
                    nidx_v)
    pltpu.sync_copy(f1g_hbm.at[pl.ds(wid * (_QPW * C), _QPW * C)], f1_v)

    rows = (rows0_v, rows1_v)
    sems = (sem0, sem1)
    lane = lax.broadcasted_iota(jnp.int32, (16,), 0)
    ngroup = NOFF_PAD // 16
    row_ids16 = [(lane + g * 16) * 16 for g in range(ngroup)]
    nchunk = C // 16

    def start(q, b):
        idx = nidx_v.at[pl.ds(q * NOFF_PAD, NOFF_PAD)]
        pltpu.async_copy(f2t_hbm.at[idx], rows[b], sems[b])

    def wait(b):
        idx = nidx_v.at[pl.ds(0, NOFF_PAD)]
        pltpu.make_async_copy(f2t_hbm.at[idx], rows[b], sems[b]).wait()

    def compute(q, b):
        rv = rows[b]
        f1c = [f1_v[pl.ds(q * C + k * 16, 16)] for k in range(nchunk)]

        def obody(i, c2):
            acc = f1c[0] * rv[i, pl.ds(0, 16)]
            for k in range(1, nchunk):
                acc = acc + f1c[k] * rv[i, pl.ds(k * 16, 16)]
            accs_v[pl.ds(i * 16, 16)] = acc
            return c2

        lax.fori_loop(0, NOFF_PAD, obody, 0)
        sums = [jnp.zeros((16,), jnp.float32) for _ in range(ngroup)]
        for l in range(16):
            for g in range(ngroup):
                sums[g] = sums[g] + plsc.load_gather(accs_v, [row_ids16[g] + l])
        for g in range(ngroup):
            out_v[pl.ds(q * NOFF_PAD + g * 16, 16)] = sums[g]

    start(0, 0)

    def pbody(p, c2):
        q0 = 2 * p
        start(q0 + 1, 1)
        wait(0)
        compute(q0, 0)
        start(q0 + 2, 0)
        wait(1)
        compute(q0 + 1, 1)
        return c2

    lax.fori_loop(0, (_QPW - 1) // 2, pbody, 0)
    wait(0)
    compute(_QPW - 1, 0)
    pltpu.sync_copy(out_v,
                    ns_hbm.at[pl.ds(wid * (_QPW * NOFF_PAD), _QPW * NOFF_PAD)])


_TW = 1024

def _tc_transpose_body(x1_ref, x2_ref, o1_ref, o2_ref):
    o1_ref[0] = x1_ref[0].T
    o2_ref[0] = x2_ref[0].T


_tc_transpose2 = pl.pallas_call(
    _tc_transpose_body,
    grid=(B, HW // _TW),
    in_specs=[
        pl.BlockSpec((1, C, _TW), lambda b, j: (b, 0, j)),
        pl.BlockSpec((1, C, _TW), lambda b, j: (b, 0, j)),
    ],
    out_specs=[
        pl.BlockSpec((1, _TW, C), lambda b, j: (b, j, 0)),
        pl.BlockSpec((1, _TW, C), lambda b, j: (b, j, 0)),
    ],
    out_shape=[
        jax.ShapeDtypeStruct((B, HW, C), jnp.float32),
        jax.ShapeDtypeStruct((B, HW, C), jnp.float32),
    ],
)


_TM, _TN = 128, 512
NSCORE = NOFF + N
_GR = (N + _TM - 1) // _TM
_GC = (NSCORE + _TN - 1) // _TN
_NCPAD = _GC * _TN

def _tc_scores_body(f1_ref, dT_ref, a_ref, b_ref, ns_ref, o_ref):
    j = pl.program_id(1)
    dots = jnp.dot(f1_ref[...], dT_ref[...],
                   preferred_element_type=jnp.float32)
    ax = a_ref[:, 0:1]
    ay = a_ref[:, 1:2]
    bx = b_ref[0:1, :]
    by = b_ref[1:2, :]
    dx = ax - bx
    dy = ay - by
    d2 = dx * dx + dy * dy
    val = jnp.where(d2 < float(NEG_D * NEG_D), 0.0, dots)
    col = lax.broadcasted_iota(jnp.int32, (_TM, _TN), 1)
    is_ns = (col < NOFF) & (j == 0)
    o_ref[...] = jnp.where(is_ns, ns_ref[...], val)


_tc_scores = pl.pallas_call(
    _tc_scores_body,
    grid=(_GR, _GC),
    in_specs=[
        pl.BlockSpec((_TM, C), lambda i, j: (i, 0)),
        pl.BlockSpec((C, _TN), lambda i, j: (0, j)),
        pl.BlockSpec((_TM, 8), lambda i, j: (i, 0)),
        pl.BlockSpec((8, _TN), lambda i, j: (0, j)),
        pl.BlockSpec((_TM, _TN), lambda i, j: (i, 0)),
    ],
    out_specs=pl.BlockSpec((_TM, _TN), lambda i, j: (i, j)),
    out_shape=jax.ShapeDtypeStruct((N, NSCORE), jnp.float32),
)


def kernel(feat1, feat2, conf1, conf2, aflow):
    k1, k2 = jax.random.split(jax.random.key(42))
    b1, y1, x1 = _rand_grid(k1)
    b3, y3, x3 = _rand_grid(k2)
    b1 = b1.astype(jnp.int32)
    b3 = b3.astype(jnp.int32)

    f1t3, f2t3 = _tc_transpose2(feat1.reshape(B, C, HW), feat2.reshape(B, C, HW))
    f1t = f1t3.reshape(B * HW, C)
    f2t = f2t3.reshape(B * HW, C)

    qidx = (b1 * HW + y1 * W + x1).astype(jnp.int32)
    didx = (b3 * HW + y3 * W + x3).astype(jnp.int32)
    pad = jnp.zeros((NPAD - N,), jnp.int32)
    qidx_p = jnp.concatenate([qidx, pad])
    didx_p = jnp.concatenate([didx, pad])

    af = aflow[b1, :, y1, x1]
    xy2 = (af + 0.5).astype(jnp.int32)
    x2, y2 = xy2[:, 0], xy2[:, 1]
    mask = ((0 <= x2) & (0 <= y2) & (x2 < W) & (y2 < H)).reshape(B, NQ)
    qconf = conf1[b1, 0, y1, x1].reshape(B, NQ)

    offx = jnp.asarray(_ALL_OFF[0])[None, :]
    offy = jnp.asarray(_ALL_OFF[1])[None, :]
    xo = jnp.clip(x2[:, None] + offx, 0, W - 1)
    yo = jnp.clip(y2[:, None] + offy, 0, H - 1)
    nidx = (b1[:, None] * HW + yo * W + xo).astype(jnp.int32)

    f1g, dg = _sc_gather_rows(f1t, f2t, qidx_p, didx_p)
    ns = _sc_ngh_scores(f2t, f1g.reshape(-1), nidx.reshape(-1))
    ns = ns.reshape(N, NOFF_PAD)

    ax = x2.astype(jnp.float32) + b1.astype(jnp.float32) * 512.0
    ay = y2.astype(jnp.float32) + b1.astype(jnp.float32) * 512.0
    bxv = x3.astype(jnp.float32) + b3.astype(jnp.float32) * 512.0
    byv = y3.astype(jnp.float32) + b3.astype(jnp.float32) * 512.0
    a_pts = jnp.zeros((_GR * _TM, 8), jnp.float32)
    a_pts = a_pts.at[:N, 0].set(ax).at[:N, 1].set(ay)
    b_pts = jnp.full((8, _NCPAD), 1e9, jnp.float32)
    b_pts = b_pts.at[0, NOFF:NOFF + N].set(bxv).at[1, NOFF:NOFF + N].set(byv)

    dTpad = jnp.zeros((C, _NCPAD), jnp.float32)
    dTpad = dTpad.at[:, NOFF:NOFF + N].set(dg[:N].T)
    ns_pad = jnp.pad(ns, ((0, 0), (0, _TN - NOFF_PAD)))

    scores = _tc_scores(f1g[:N], dTpad, a_pts, b_pts, ns_pad)
    gt = jnp.concatenate([
        jnp.ones((N, NPOS), jnp.uint8),
        jnp.zeros((N, NOFF - NPOS + N), jnp.uint8),
    ], axis=1)
    return (scores, gt, mask, qconf)


# speedup vs baseline: 1.5123x; 1.5123x over previous
import functools

import numpy as np
import jax
import jax.numpy as jnp
from jax import lax
from jax.experimental import pallas as pl
from jax.experimental.pallas import tpu as pltpu
from jax.experimental.pallas import tpu_sc as plsc

NGH = 7
SUBQ = -8
SUBD = 1
POS_D = 3
NEG_D = 5
BORDER = 16
SUBD_NEG = -8
B, C, H, W = 2, 128, 384, 384
HW = H * W

def _make_offsets():
    pos, neg = [], []
    pos_d2, neg_d2, rad2 = POS_D ** 2, NEG_D ** 2, NGH ** 2
    rad = (NGH // SUBD) * NGH
    for j in range(-rad, rad + 1, SUBD):
        for i in range(-rad, rad + 1, SUBD):
            d2 = i * i + j * j
            if d2 <= pos_d2:
                pos.append((i, j))
            elif neg_d2 <= d2 <= rad2:
                neg.append((i, j))
    return (np.array(pos, dtype=np.int32).T, np.array(neg, dtype=np.int32).T)

_POS_OFF, _NEG_OFF = _make_offsets()
NPOS = _POS_OFF.shape[1]
NNEG = _NEG_OFF.shape[1]
NOFF = NPOS + NNEG
NOFF_PAD = 112
_ALL_OFF = np.concatenate(
    [_POS_OFF, _NEG_OFF, np.zeros((2, NOFF_PAD - NOFF), np.int32)], axis=1)

NQ = (H - 2 * BORDER) * (W - 2 * BORDER) // (SUBQ * SUBQ)
N = B * NQ
NPAD = 4096

_NC, _NS = 2, 16
_NW = _NC * _NS
_QPW = N // _NW
_GPW = NPAD // _NW

_sc_mesh = plsc.VectorSubcoreMesh(
    core_axis_name="c", subcore_axis_name="s", num_cores=_NC, num_subcores=_NS)


def _rand_grid(key):
    kx, ky = jax.random.split(key)
    x = jax.random.randint(kx, (NQ,), BORDER, W - BORDER)
    y = jax.random.randint(ky, (NQ,), BORDER, H - BORDER)
    x = jnp.broadcast_to(x[None, :], (B, NQ)).reshape(-1)
    y = jnp.broadcast_to(y[None, :], (B, NQ)).reshape(-1)
    b = jnp.broadcast_to(jnp.arange(B)[:, None], (B, NQ)).reshape(-1)
    return b, y, x


@functools.partial(
    pl.kernel,
    out_type=jax.ShapeDtypeStruct((N * NOFF_PAD,), jnp.float32),
    mesh=_sc_mesh,
    compiler_params=pltpu.CompilerParams(needs_layout_passes=False),
    scratch_types=[
        pltpu.VMEM((_QPW * NOFF_PAD,), jnp.int32),
        pltpu.VMEM((_QPW * C,), jnp.float32),
        pltpu.VMEM((NOFF_PAD, C), jnp.float32),
        pltpu.VMEM((NOFF_PAD, C), jnp.float32),
        pltpu.VMEM((NOFF_PAD * 16,), jnp.float32),
        pltpu.VMEM((_QPW * NOFF_PAD,), jnp.float32),
        pltpu.SemaphoreType.DMA,
        pltpu.SemaphoreType.DMA,
    ],
)
def _sc_ngh_scores(f2t_hbm, f1g_hbm, nidx_hbm, ns_hbm,
                   nidx_v, f1_v, rows0_v, rows1_v, accs_v, out_v, sem0, sem1):
    wid = lax.axis_index("s") * _NC + lax.axis_index("c")

    pltpu.sync_copy(nidx_hbm.at[pl.ds(wid * (_QPW * NOFF_PAD), _QPW * NOFF_PAD)],
                    nidx_v)
    pltpu.sync_copy(f1g_hbm.at[pl.ds(wid * (_QPW * C), _QPW * C)], f1_v)

    rows = (rows0_v, rows1_v)
    sems = (sem0, sem1)
    lane = lax.broadcasted_iota(jnp.int32, (16,), 0)
    ngroup = NOFF_PAD // 16
    row_ids16 = [(lane + g * 16) * 16 for g in range(ngroup)]
    nchunk = C // 16

    def start(q, b):
        idx = nidx_v.at[pl.ds(q * NOFF_PAD, NOFF_PAD)]
        pltpu.async_copy(f2t_hbm.at[idx], rows[b], sems[b])

    def wait(b):
        idx = nidx_v.at[pl.ds(0, NOFF_PAD)]
        pltpu.make_async_copy(f2t_hbm.at[idx], rows[b], sems[b]).wait()

    def compute(q, b):
        rv = rows[b]
        f1c = [f1_v[pl.ds(q * C + k * 16, 16)] for k in range(nchunk)]

        def obody(i, c2):
            acc = f1c[0] * rv[i, pl.ds(0, 16)]
            for k in range(1, nchunk):
                acc = acc + f1c[k] * rv[i, pl.ds(k * 16, 16)]
            accs_v[pl.ds(i * 16, 16)] = acc
            return c2

        lax.fori_loop(0, NOFF_PAD, obody, 0)
        sums = [jnp.zeros((16,), jnp.float32) for _ in range(ngroup)]
        for l in range(16):
            for g in range(ngroup):
                sums[g] = sums[g] + plsc.load_gather(accs_v, [row_ids16[g] + l])
        for g in range(ngroup):
            out_v[pl.ds(q * NOFF_PAD + g * 16, 16)] = sums[g]

    start(0, 0)

    def pbody(p, c2):
        q0 = 2 * p
        start(q0 + 1, 1)
        wait(0)
        compute(q0, 0)
        start(q0 + 2, 0)
        wait(1)
        compute(q0 + 1, 1)
        return c2

    lax.fori_loop(0, (_QPW - 1) // 2, pbody, 0)
    wait(0)
    compute(_QPW - 1, 0)
    pltpu.sync_copy(out_v,
                    ns_hbm.at[pl.ds(wid * (_QPW * NOFF_PAD), _QPW * NOFF_PAD)])


_TM, _TN = 128, 512
NSCORE = NOFF + N
_GR = (N + _TM - 1) // _TM
_GC = (NSCORE + _TN - 1) // _TN
_NCPAD = _GC * _TN

def _tc_scores_body(f1_ref, dT_ref, a_ref, b_ref, ns_ref, o_ref):
    j = pl.program_id(1)
    dots = jnp.dot(f1_ref[...], dT_ref[...],
                   preferred_element_type=jnp.float32)
    ax = a_ref[:, 0:1]
    ay = a_ref[:, 1:2]
    bx = b_ref[0:1, :]
    by = b_ref[1:2, :]
    dx = ax - bx
    dy = ay - by
    d2 = dx * dx + dy * dy
    val = jnp.where(d2 < float(NEG_D * NEG_D), 0.0, dots)
    col = lax.broadcasted_iota(jnp.int32, (_TM, _TN), 1)
    is_ns = (col < NOFF) & (j == 0)
    o_ref[...] = jnp.where(is_ns, ns_ref[...], val)


_tc_scores = pl.pallas_call(
    _tc_scores_body,
    grid=(_GR, _GC),
    in_specs=[
        pl.BlockSpec((_TM, C), lambda i, j: (i, 0)),
        pl.BlockSpec((C, _TN), lambda i, j: (0, j)),
        pl.BlockSpec((_TM, 8), lambda i, j: (i, 0)),
        pl.BlockSpec((8, _TN), lambda i, j: (0, j)),
        pl.BlockSpec((_TM, _TN), lambda i, j: (i, 0)),
    ],
    out_specs=pl.BlockSpec((_TM, _TN), lambda i, j: (i, j)),
    out_shape=jax.ShapeDtypeStruct((N, NSCORE), jnp.float32),
)


def kernel(feat1, feat2, conf1, conf2, aflow):
    k1, k2 = jax.random.split(jax.random.key(42))
    b1, y1, x1 = _rand_grid(k1)
    b3, y3, x3 = _rand_grid(k2)
    b1 = b1.astype(jnp.int32)
    b3 = b3.astype(jnp.int32)

    f2t = feat2.transpose(0, 2, 3, 1).reshape(B * HW, C)

    f1g = feat1[b1, :, y1, x1]
    dg = feat2[b3, :, y3, x3]

    af = aflow[b1, :, y1, x1]
    xy2 = (af + 0.5).astype(jnp.int32)
    x2, y2 = xy2[:, 0], xy2[:, 1]
    mask = ((0 <= x2) & (0 <= y2) & (x2 < W) & (y2 < H)).reshape(B, NQ)
    qconf = conf1[b1, 0, y1, x1].reshape(B, NQ)

    offx = jnp.asarray(_ALL_OFF[0])[None, :]
    offy = jnp.asarray(_ALL_OFF[1])[None, :]
    xo = jnp.clip(x2[:, None] + offx, 0, W - 1)
    yo = jnp.clip(y2[:, None] + offy, 0, H - 1)
    nidx = (b1[:, None] * HW + yo * W + xo).astype(jnp.int32)

    ns = _sc_ngh_scores(f2t, f1g.reshape(-1), nidx.reshape(-1))
    ns = ns.reshape(N, NOFF_PAD)

    ax = x2.astype(jnp.float32) + b1.astype(jnp.float32) * 512.0
    ay = y2.astype(jnp.float32) + b1.astype(jnp.float32) * 512.0
    bxv = x3.astype(jnp.float32) + b3.astype(jnp.float32) * 512.0
    byv = y3.astype(jnp.float32) + b3.astype(jnp.float32) * 512.0
    a_pts = jnp.zeros((_GR * _TM, 8), jnp.float32)
    a_pts = a_pts.at[:N, 0].set(ax).at[:N, 1].set(ay)
    b_pts = jnp.full((8, _NCPAD), 1e9, jnp.float32)
    b_pts = b_pts.at[0, NOFF:NOFF + N].set(bxv).at[1, NOFF:NOFF + N].set(byv)

    dTpad = jnp.zeros((C, _NCPAD), jnp.float32)
    dTpad = dTpad.at[:, NOFF:NOFF + N].set(dg.T)
    ns_pad = jnp.pad(ns, ((0, 0), (0, _TN - NOFF_PAD)))

    scores = _tc_scores(f1g, dTpad, a_pts, b_pts, ns_pad)
    gt = jnp.concatenate([
        jnp.ones((N, NPOS), jnp.uint8),
        jnp.zeros((N, NOFF - NPOS + N), jnp.uint8),
    ], axis=1)
    return (scores, gt, mask, qconf)


# speedup vs baseline: 1.7839x; 1.1796x over previous
import functools

import numpy as np
import jax
import jax.numpy as jnp
from jax import lax
from jax.experimental import pallas as pl
from jax.experimental.pallas import tpu as pltpu
from jax.experimental.pallas import tpu_sc as plsc

NGH = 7
SUBQ = -8
SUBD = 1
POS_D = 3
NEG_D = 5
BORDER = 16
SUBD_NEG = -8
B, C, H, W = 2, 128, 384, 384
HW = H * W

def _make_offsets():
    pos, neg = [], []
    pos_d2, neg_d2, rad2 = POS_D ** 2, NEG_D ** 2, NGH ** 2
    rad = (NGH // SUBD) * NGH
    for j in range(-rad, rad + 1, SUBD):
        for i in range(-rad, rad + 1, SUBD):
            d2 = i * i + j * j
            if d2 <= pos_d2:
                pos.append((i, j))
            elif neg_d2 <= d2 <= rad2:
                neg.append((i, j))
    return (np.array(pos, dtype=np.int32).T, np.array(neg, dtype=np.int32).T)

_POS_OFF, _NEG_OFF = _make_offsets()
NPOS = _POS_OFF.shape[1]
NNEG = _NEG_OFF.shape[1]
NOFF = NPOS + NNEG
NOFF_PAD = 112
_ALL_OFF = np.concatenate(
    [_POS_OFF, _NEG_OFF, np.zeros((2, NOFF_PAD - NOFF), np.int32)], axis=1)

NQ = (H - 2 * BORDER) * (W - 2 * BORDER) // (SUBQ * SUBQ)
N = B * NQ
NPAD = 4096

_NC, _NS = 2, 16
_NW = _NC * _NS
_QPW = N // _NW
_GPW = NPAD // _NW

_sc_mesh = plsc.VectorSubcoreMesh(
    core_axis_name="c", subcore_axis_name="s", num_cores=_NC, num_subcores=_NS)


def _rand_grid(key):
    kx, ky = jax.random.split(key)
    x = jax.random.randint(kx, (NQ,), BORDER, W - BORDER)
    y = jax.random.randint(ky, (NQ,), BORDER, H - BORDER)
    x = jnp.broadcast_to(x[None, :], (B, NQ)).reshape(-1)
    y = jnp.broadcast_to(y[None, :], (B, NQ)).reshape(-1)
    b = jnp.broadcast_to(jnp.arange(B)[:, None], (B, NQ)).reshape(-1)
    return b, y, x


@functools.partial(
    pl.kernel,
    out_type=jax.ShapeDtypeStruct((N * NOFF_PAD,), jnp.float32),
    mesh=_sc_mesh,
    compiler_params=pltpu.CompilerParams(needs_layout_passes=False),
    scratch_types=[
        pltpu.VMEM((_QPW * NOFF_PAD,), jnp.int32),
        pltpu.VMEM((_QPW * C,), jnp.float32),
        pltpu.VMEM((NOFF_PAD, C), jnp.float32),
        pltpu.VMEM((NOFF_PAD, C), jnp.float32),
        pltpu.VMEM((NOFF_PAD * 16,), jnp.float32),
        pltpu.VMEM((_QPW * NOFF_PAD,), jnp.float32),
        pltpu.SemaphoreType.DMA,
        pltpu.SemaphoreType.DMA,
    ],
)
def _sc_ngh_scores(f2t_hbm, f1g_hbm, nidx_hbm, ns_hbm,
                   nidx_v, f1_v, rows0_v, rows1_v, accs_v, out_v, sem0, sem1):
    wid = lax.axis_index("s") * _NC + lax.axis_index("c")

    pltpu.sync_copy(nidx_hbm.at[pl.ds(wid * (_QPW * NOFF_PAD), _QPW * NOFF_PAD)],
                    nidx_v)
    pltpu.sync_copy(f1g_hbm.at[pl.ds(wid * (_QPW * C), _QPW * C)], f1_v)

    rows = (rows0_v, rows1_v)
    sems = (sem0, sem1)
    lane = lax.broadcasted_iota(jnp.int32, (16,), 0)
    ngroup = NOFF_PAD // 16
    row_ids16 = [(lane + g * 16) * 16 for g in range(ngroup)]
    nchunk = C // 16

    def start(q, b):
        idx = nidx_v.at[pl.ds(q * NOFF_PAD, NOFF_PAD)]
        pltpu.async_copy(f2t_hbm.at[idx], rows[b], sems[b])

    def wait(b):
        idx = nidx_v.at[pl.ds(0, NOFF_PAD)]
        pltpu.make_async_copy(f2t_hbm.at[idx], rows[b], sems[b]).wait()

    def compute(q, b):
        rv = rows[b]
        f1c = [f1_v[pl.ds(q * C + k * 16, 16)] for k in range(nchunk)]

        def obody(i, c2):
            acc = f1c[0] * rv[i, pl.ds(0, 16)]
            for k in range(1, nchunk):
                acc = acc + f1c[k] * rv[i, pl.ds(k * 16, 16)]
            accs_v[pl.ds(i * 16, 16)] = acc
            return c2

        lax.fori_loop(0, NOFF_PAD, obody, 0)
        sums = [jnp.zeros((16,), jnp.float32) for _ in range(ngroup)]
        for l in range(16):
            for g in range(ngroup):
                sums[g] = sums[g] + plsc.load_gather(accs_v, [row_ids16[g] + l])
        for g in range(ngroup):
            out_v[pl.ds(q * NOFF_PAD + g * 16, 16)] = sums[g]

    start(0, 0)

    def pbody(p, c2):
        q0 = 2 * p
        start(q0 + 1, 1)
        wait(0)
        compute(q0, 0)
        start(q0 + 2, 0)
        wait(1)
        compute(q0 + 1, 1)
        return c2

    lax.fori_loop(0, (_QPW - 1) // 2, pbody, 0)
    wait(0)
    compute(_QPW - 1, 0)
    pltpu.sync_copy(out_v,
                    ns_hbm.at[pl.ds(wid * (_QPW * NOFF_PAD), _QPW * NOFF_PAD)])


_TM, _TN = 128, 512
NSCORE = NOFF + N
_GR = (N + _TM - 1) // _TM
_GC = (NSCORE + _TN - 1) // _TN
_NCPAD = _GC * _TN

def _tc_scores_body(f1_ref, dT_ref, a_ref, b_ref, ns_ref, o_ref):
    dots = jnp.dot(f1_ref[...], dT_ref[...],
                   preferred_element_type=jnp.float32)
    ax = a_ref[:, 0:1]
    ay = a_ref[:, 1:2]
    bx = b_ref[0:1, :]
    by = b_ref[1:2, :]
    dx = ax - bx
    dy = ay - by
    d2 = dx * dx + dy * dy
    val = jnp.where(d2 < float(NEG_D * NEG_D), 0.0, dots)
    col = lax.broadcasted_iota(jnp.int32, (_TM, _TN), 1)
    o_ref[:, : _TN] = jnp.where(col < NOFF, ns_ref[...], val[:, : _TN])
    o_ref[:, _TN:] = val[:, _TN:]


_tc_scores = pl.pallas_call(
    _tc_scores_body,
    grid=(_GR,),
    in_specs=[
        pl.BlockSpec((_TM, C), lambda i: (i, 0)),
        pl.BlockSpec((C, _NCPAD), lambda i: (0, 0)),
        pl.BlockSpec((_TM, 8), lambda i: (i, 0)),
        pl.BlockSpec((8, _NCPAD), lambda i: (0, 0)),
        pl.BlockSpec((_TM, _TN), lambda i: (i, 0)),
    ],
    out_specs=pl.BlockSpec((_TM, _NCPAD), lambda i: (i, 0)),
    out_shape=jax.ShapeDtypeStruct((N, NSCORE), jnp.float32),
)


def kernel(feat1, feat2, conf1, conf2, aflow):
    k1, k2 = jax.random.split(jax.random.key(42))
    b1, y1, x1 = _rand_grid(k1)
    b3, y3, x3 = _rand_grid(k2)
    b1 = b1.astype(jnp.int32)
    b3 = b3.astype(jnp.int32)

    f2t = feat2.transpose(0, 2, 3, 1).reshape(B * HW, C)

    f1g = feat1[b1, :, y1, x1]
    dg = feat2[b3, :, y3, x3]

    af = aflow[b1, :, y1, x1]
    xy2 = (af + 0.5).astype(jnp.int32)
    x2, y2 = xy2[:, 0], xy2[:, 1]
    mask = ((0 <= x2) & (0 <= y2) & (x2 < W) & (y2 < H)).reshape(B, NQ)
    qconf = conf1[b1, 0, y1, x1].reshape(B, NQ)

    offx = jnp.asarray(_ALL_OFF[0])[None, :]
    offy = jnp.asarray(_ALL_OFF[1])[None, :]
    xo = jnp.clip(x2[:, None] + offx, 0, W - 1)
    yo = jnp.clip(y2[:, None] + offy, 0, H - 1)
    nidx = (b1[:, None] * HW + yo * W + xo).astype(jnp.int32)

    ns = _sc_ngh_scores(f2t, f1g.reshape(-1), nidx.reshape(-1))
    ns = ns.reshape(N, NOFF_PAD)

    ax = x2.astype(jnp.float32) + b1.astype(jnp.float32) * 512.0
    ay = y2.astype(jnp.float32) + b1.astype(jnp.float32) * 512.0
    bxv = x3.astype(jnp.float32) + b3.astype(jnp.float32) * 512.0
    byv = y3.astype(jnp.float32) + b3.astype(jnp.float32) * 512.0
    a_pts = jnp.zeros((_GR * _TM, 8), jnp.float32)
    a_pts = a_pts.at[:N, 0].set(ax).at[:N, 1].set(ay)
    b_pts = jnp.full((8, _NCPAD), 1e9, jnp.float32)
    b_pts = b_pts.at[0, NOFF:NOFF + N].set(bxv).at[1, NOFF:NOFF + N].set(byv)

    dTpad = jnp.zeros((C, _NCPAD), jnp.float32)
    dTpad = dTpad.at[:, NOFF:NOFF + N].set(dg.T)
    ns_pad = jnp.pad(ns, ((0, 0), (0, _TN - NOFF_PAD)))

    scores = _tc_scores(f1g, dTpad, a_pts, b_pts, ns_pad)
    gt = jnp.concatenate([
        jnp.ones((N, NPOS), jnp.uint8),
        jnp.zeros((N, NOFF - NPOS + N), jnp.uint8),
    ], axis=1)
    return (scores, gt, mask, qconf)
